# baseline (speedup 1.0000x reference)
import functools
import math

import jax
import jax.numpy as jnp
from jax import lax
from jax.experimental import pallas as pl
from jax.experimental.pallas import tpu as pltpu

TOPK = 8
LT = 256
SBLK = 5120
NEG = float("-inf")


def _pool_q_kernel(x_ref, wk_ref, bk_ref, q_ref, acc_s):
    i = pl.program_id(0)

    @pl.when(i == 0)
    def _():
        acc_s[...] = jnp.zeros_like(acc_s)

    b = x_ref.shape[0]
    acc_s[0:b, :] += jnp.sum(x_ref[...], axis=1)

    @pl.when(i == pl.num_programs(0) - 1)
    def _():
        nl = pl.num_programs(0) * x_ref.shape[1]
        xbar = acc_s[0:b, :] * (1.0 / nl)
        q_ref[...] = (
            jnp.dot(xbar, wk_ref[...], preferred_element_type=jnp.float32,
                    precision=lax.Precision.HIGHEST)
            + bk_ref[...]
        )


def _pool_q(x, W_key, b_key2):
    B, L, D = x.shape
    KD = W_key.shape[1]
    return pl.pallas_call(
        _pool_q_kernel,
        grid=(L // LT,),
        in_specs=[
            pl.BlockSpec((B, LT, D), lambda i: (0, i, 0)),
            pl.BlockSpec((D, KD), lambda i: (0, 0)),
            pl.BlockSpec((1, KD), lambda i: (0, 0)),
        ],
        out_specs=pl.BlockSpec((B, KD), lambda i: (0, 0)),
        out_shape=jax.ShapeDtypeStruct((B, KD), jnp.float32),
        scratch_shapes=[pltpu.VMEM((8, D), jnp.float32)],
    )(x, W_key, b_key2)


def _topk_kernel(s_total, q_ref, ck_ref, ti_ref, w_ref, sims_s):
    i = pl.program_id(0)
    B = q_ref.shape[0]
    sblk = ck_ref.shape[0]

    s_blk = lax.dot_general(
        q_ref[...], ck_ref[...],
        dimension_numbers=(((1,), (1,)), ((), ())),
        preferred_element_type=jnp.float32,
        precision=lax.Precision.HIGHEST,
    )
    gidx = i * sblk + lax.broadcasted_iota(jnp.int32, (B, sblk), 1)
    s_blk = jnp.where(gidx < s_total, s_blk, NEG)
    sims_s[:, pl.ds(i * sblk, sblk)] = s_blk

    @pl.when(i == pl.num_programs(0) - 1)
    def _():
        s = sims_s[...]
        n_pad = s.shape[1]
        iota = lax.broadcasted_iota(jnp.int32, (B, n_pad), 1)
        nv, ni = [], []
        for _ in range(TOPK):
            m = jnp.max(s, axis=1, keepdims=True)
            pos = jnp.min(jnp.where(s == m, iota, n_pad), axis=1,
                          keepdims=True)
            nv.append(m)
            ni.append(pos)
            s = jnp.where(iota == pos, NEG, s)
        tv = jnp.concatenate(nv, axis=1)
        ti = jnp.concatenate(ni, axis=1)
        e = jnp.exp(tv - tv[:, 0:1])
        w = e / jnp.sum(e, axis=1, keepdims=True)
        conf = jnp.mean(w[:, 0:1])
        ti_ref[0:B, 0:TOPK] = ti
        w_ref[0:B, 0:TOPK] = w
        w_ref[B:B + 1, 0:1] = jnp.reshape(conf, (1, 1))


def _sims_topk(q, cube_keys):
    B = q.shape[0]
    S, KD = cube_keys.shape
    nblk = (S + SBLK - 1) // SBLK
    return pl.pallas_call(
        functools.partial(_topk_kernel, S),
        grid=(nblk,),
        in_specs=[
            pl.BlockSpec((B, KD), lambda i: (0, 0)),
            pl.BlockSpec((SBLK, KD), lambda i: (i, 0)),
        ],
        out_specs=[
            pl.BlockSpec((8, 128), lambda i: (0, 0)),
            pl.BlockSpec((8, 128), lambda i: (0, 0)),
        ],
        out_shape=[
            jax.ShapeDtypeStruct((8, 128), jnp.int32),
            jax.ShapeDtypeStruct((8, 128), jnp.float32),
        ],
        scratch_shapes=[
            pltpu.VMEM((B, nblk * SBLK), jnp.float32),
        ],
    )(q, cube_keys)


def _gather_rows(cube_values, idx):
    from jax.experimental.pallas import tpu_sc as plsc

    n = idx.shape[0]
    VD = cube_values.shape[1]
    per = 8
    nw = n // per
    mesh = plsc.VectorSubcoreMesh(core_axis_name="c", subcore_axis_name="s")

    @functools.partial(
        pl.kernel,
        mesh=mesh,
        out_type=jax.ShapeDtypeStruct((n, VD), jnp.float32),
        scratch_types=[
            pltpu.VMEM((per,), jnp.int32),
            pltpu.VMEM((per, VD), jnp.float32),
            pltpu.SemaphoreType.DMA,
        ],
    )
    def k(cv_hbm, idx_hbm, out_hbm, idx_v, rows_v, sem):
        wid = lax.axis_index("c") * 16 + lax.axis_index("s")

        @pl.when(wid < nw)
        def _():
            base = wid * per
            pltpu.sync_copy(idx_hbm.at[pl.ds(base, per)], idx_v)
            pltpu.async_copy(cv_hbm.at[idx_v], rows_v, sem).wait()
            pltpu.sync_copy(rows_v, out_hbm.at[pl.ds(base, per)])

    return k(cube_values, idx)


def _main_kernel(x_ref, g_ref, w_ref, conf_ref, wm_ref, bm_ref, wg1_ref,
                 wrow_ref, bg1_ref, wg2_ref, bg2_ref, lng_ref, lnb_ref,
                 out_ref):
    xb = x_ref[0]
    conf = conf_ref[0, 0]

    mv = jnp.dot(w_ref[0], g_ref[0], preferred_element_type=jnp.float32)
    mem = (
        jnp.dot(mv, wm_ref[...], preferred_element_type=jnp.float32)
        + bm_ref[...]
    )

    h = (
        jnp.dot(xb, wg1_ref[...], preferred_element_type=jnp.float32)
        + conf * wrow_ref[...]
        + bg1_ref[...]
    )
    h = 0.5 * h * (1.0 + lax.erf(h * (1.0 / math.sqrt(2.0))))
    ap = jnp.sum(h * wg2_ref[...], axis=1, keepdims=True) + bg2_ref[...]
    alpha = jax.nn.sigmoid(ap)

    y = xb + (1.0 - alpha) * mem
    mu = jnp.mean(y, axis=1, keepdims=True)
    yc = y - mu
    var = jnp.mean(yc * yc, axis=1, keepdims=True)
    out_ref[0] = yc * lax.rsqrt(var + 1e-5) * lng_ref[...] + lnb_ref[...]


def _main(x, g, w3, conf2, W_mem, bm2, Wg1a, wrow, bg12, wg2r, bg22,
          lng2, lnb2):
    B, L, D = x.shape
    K, VD = g.shape[1], g.shape[2]
    H = Wg1a.shape[1]
    return pl.pallas_call(
        _main_kernel,
        grid=(B, L // LT),
        in_specs=[
            pl.BlockSpec((1, LT, D), lambda b, l: (b, l, 0)),
            pl.BlockSpec((1, K, VD), lambda b, l: (b, 0, 0)),
            pl.BlockSpec((1, 1, K), lambda b, l: (b, 0, 0)),
            pl.BlockSpec((1, 1), lambda b, l: (0, 0)),
            pl.BlockSpec((VD, D), lambda b, l: (0, 0)),
            pl.BlockSpec((1, D), lambda b, l: (0, 0)),
            pl.BlockSpec((D, H), lambda b, l: (0, 0)),
            pl.BlockSpec((1, H), lambda b, l: (0, 0)),
            pl.BlockSpec((1, H), lambda b, l: (0, 0)),
            pl.BlockSpec((1, H), lambda b, l: (0, 0)),
            pl.BlockSpec((1, 1), lambda b, l: (0, 0)),
            pl.BlockSpec((1, D), lambda b, l: (0, 0)),
            pl.BlockSpec((1, D), lambda b, l: (0, 0)),
        ],
        out_specs=pl.BlockSpec((1, LT, D), lambda b, l: (b, l, 0)),
        out_shape=jax.ShapeDtypeStruct((B, L, D), jnp.float32),
    )(x, g, w3, conf2, W_mem, bm2, Wg1a, wrow, bg12, wg2r, bg22, lng2, lnb2)


def kernel(x, W_key, b_key, cube_keys, cube_values, W_mem, b_mem,
           Wg1, bg1, Wg2, bg2, ln_g, ln_b):
    B, L, D = x.shape
    KD = W_key.shape[1]
    VD = cube_values.shape[1]
    H = Wg1.shape[1]

    q = _pool_q(x, W_key, b_key.reshape(1, KD))
    ti_p, w_p = _sims_topk(q, cube_keys)
    idx = ti_p[:B, :TOPK].reshape(B * TOPK)
    gathered = _gather_rows(cube_values, idx)

    out = _main(
        x,
        gathered.reshape(B, TOPK, VD),
        w_p[:B, :TOPK].reshape(B, 1, TOPK),
        w_p[B:B + 1, 0:1],
        W_mem,
        b_mem.reshape(1, D),
        Wg1[:D],
        Wg1[D:D + 1],
        bg1.reshape(1, H),
        Wg2.reshape(1, H),
        bg2.reshape(1, 1),
        ln_g.reshape(1, D),
        ln_b.reshape(1, D),
    )
    return out


# speedup vs baseline: 1.0832x; 1.0832x over previous
import functools
import math

import jax
import jax.numpy as jnp
from jax import lax
from jax.experimental import pallas as pl
from jax.experimental.pallas import tpu as pltpu

TOPK = 8
LT = 256
SBLK = 5120
NEG = float("-inf")


def _pool_q_kernel(x_ref, wk_ref, bk_ref, q_ref, acc_s):
    i = pl.program_id(0)

    @pl.when(i == 0)
    def _():
        acc_s[...] = jnp.zeros_like(acc_s)

    b = x_ref.shape[0]
    xr = x_ref[...].astype(jnp.bfloat16).astype(jnp.float32)
    acc_s[0:b, :] += jnp.sum(xr, axis=1)

    @pl.when(i == pl.num_programs(0) - 1)
    def _():
        nl = pl.num_programs(0) * x_ref.shape[1]
        xbar = acc_s[0:b, :] * (1.0 / nl)
        wkr = wk_ref[...].astype(jnp.bfloat16).astype(jnp.float32)
        q_ref[...] = (
            jnp.dot(xbar, wkr, preferred_element_type=jnp.float32,
                    precision=lax.Precision.HIGHEST)
            + bk_ref[...]
        )


def _pool_q(x, W_key, b_key2):
    B, L, D = x.shape
    KD = W_key.shape[1]
    return pl.pallas_call(
        _pool_q_kernel,
        grid=(L // LT,),
        in_specs=[
            pl.BlockSpec((B, LT, D), lambda i: (0, i, 0)),
            pl.BlockSpec((D, KD), lambda i: (0, 0)),
            pl.BlockSpec((1, KD), lambda i: (0, 0)),
        ],
        out_specs=pl.BlockSpec((B, KD), lambda i: (0, 0)),
        out_shape=jax.ShapeDtypeStruct((B, KD), jnp.float32),
        scratch_shapes=[pltpu.VMEM((8, D), jnp.float32)],
    )(x, W_key, b_key2)


def _topk_kernel(s_total, q_ref, ck_ref, ti_ref, w_ref, sims_s):
    i = pl.program_id(0)
    B = q_ref.shape[0]
    sblk = ck_ref.shape[0]

    s_blk = lax.dot_general(
        q_ref[...].astype(jnp.bfloat16), ck_ref[...].astype(jnp.bfloat16),
        dimension_numbers=(((1,), (1,)), ((), ())),
        preferred_element_type=jnp.float32,
    )
    gidx = i * sblk + lax.broadcasted_iota(jnp.int32, (B, sblk), 1)
    s_blk = jnp.where(gidx < s_total, s_blk, NEG)
    sims_s[:, pl.ds(i * sblk, sblk)] = s_blk

    @pl.when(i == pl.num_programs(0) - 1)
    def _():
        s = sims_s[...]
        n_pad = s.shape[1]
        iota = lax.broadcasted_iota(jnp.int32, (B, n_pad), 1)
        nv, ni = [], []
        for _ in range(TOPK):
            m = jnp.max(s, axis=1, keepdims=True)
            pos = jnp.min(jnp.where(s == m, iota, n_pad), axis=1,
                          keepdims=True)
            nv.append(m)
            ni.append(pos)
            s = jnp.where(iota == pos, NEG, s)
        tv = jnp.concatenate(nv, axis=1)
        ti = jnp.concatenate(ni, axis=1)
        e = jnp.exp(tv - tv[:, 0:1])
        w = e / jnp.sum(e, axis=1, keepdims=True)
        conf = jnp.mean(w[:, 0:1])
        ti_ref[0:B, 0:TOPK] = ti
        w_ref[0:B, 0:TOPK] = w
        w_ref[B:B + 1, 0:1] = jnp.reshape(conf, (1, 1))


def _sims_topk(q, cube_keys):
    B = q.shape[0]
    S, KD = cube_keys.shape
    nblk = (S + SBLK - 1) // SBLK
    return pl.pallas_call(
        functools.partial(_topk_kernel, S),
        grid=(nblk,),
        in_specs=[
            pl.BlockSpec((B, KD), lambda i: (0, 0)),
            pl.BlockSpec((SBLK, KD), lambda i: (i, 0)),
        ],
        out_specs=[
            pl.BlockSpec((8, 128), lambda i: (0, 0)),
            pl.BlockSpec((8, 128), lambda i: (0, 0)),
        ],
        out_shape=[
            jax.ShapeDtypeStruct((8, 128), jnp.int32),
            jax.ShapeDtypeStruct((8, 128), jnp.float32),
        ],
        scratch_shapes=[
            pltpu.VMEM((B, nblk * SBLK), jnp.float32),
        ],
    )(q, cube_keys)


def _gather_rows(cube_values, idx):
    from jax.experimental.pallas import tpu_sc as plsc

    n = idx.shape[0]
    VD = cube_values.shape[1]
    per = 8
    nw = n // per
    mesh = plsc.VectorSubcoreMesh(core_axis_name="c", subcore_axis_name="s")

    @functools.partial(
        pl.kernel,
        mesh=mesh,
        out_type=jax.ShapeDtypeStruct((n, VD), jnp.float32),
        scratch_types=[
            pltpu.VMEM((per,), jnp.int32),
            pltpu.VMEM((per, VD), jnp.float32),
            pltpu.SemaphoreType.DMA,
        ],
    )
    def k(cv_hbm, idx_hbm, out_hbm, idx_v, rows_v, sem):
        wid = lax.axis_index("c") * 16 + lax.axis_index("s")

        @pl.when(wid < nw)
        def _():
            base = wid * per
            pltpu.sync_copy(idx_hbm.at[pl.ds(base, per)], idx_v)
            pltpu.async_copy(cv_hbm.at[idx_v], rows_v, sem).wait()
            pltpu.sync_copy(rows_v, out_hbm.at[pl.ds(base, per)])

    return k(cube_values, idx)


def _main_kernel(x_ref, g_ref, w_ref, conf_ref, wm_ref, bm_ref, wg1_ref,
                 wrow_ref, bg1_ref, wg2_ref, bg2_ref, lng_ref, lnb_ref,
                 out_ref):
    xb = x_ref[0]
    conf = conf_ref[0, 0]

    mv = jnp.dot(w_ref[0], g_ref[0], preferred_element_type=jnp.float32,
                 precision=lax.Precision.HIGHEST)
    mem = (
        jnp.dot(mv, wm_ref[...], preferred_element_type=jnp.float32,
                precision=lax.Precision.HIGHEST)
        + bm_ref[...]
    )

    h = (
        jnp.dot(xb.astype(jnp.bfloat16),
                wg1_ref[...].astype(jnp.bfloat16),
                preferred_element_type=jnp.float32)
        + conf * wrow_ref[...]
        + bg1_ref[...]
    )
    h = 0.5 * h * (1.0 + lax.erf(h * (1.0 / math.sqrt(2.0))))
    ap = jnp.sum(h * wg2_ref[...], axis=1, keepdims=True) + bg2_ref[...]
    alpha = jax.nn.sigmoid(ap)

    y = xb + (1.0 - alpha) * mem
    mu = jnp.mean(y, axis=1, keepdims=True)
    yc = y - mu
    var = jnp.mean(yc * yc, axis=1, keepdims=True)
    out_ref[0] = yc * lax.rsqrt(var + 1e-5) * lng_ref[...] + lnb_ref[...]


def _main(x, g, w3, conf2, W_mem, bm2, Wg1a, wrow, bg12, wg2r, bg22,
          lng2, lnb2):
    B, L, D = x.shape
    K, VD = g.shape[1], g.shape[2]
    H = Wg1a.shape[1]
    return pl.pallas_call(
        _main_kernel,
        grid=(B, L // LT),
        in_specs=[
            pl.BlockSpec((1, LT, D), lambda b, l: (b, l, 0)),
            pl.BlockSpec((1, K, VD), lambda b, l: (b, 0, 0)),
            pl.BlockSpec((1, 1, K), lambda b, l: (b, 0, 0)),
            pl.BlockSpec((1, 1), lambda b, l: (0, 0)),
            pl.BlockSpec((VD, D), lambda b, l: (0, 0)),
            pl.BlockSpec((1, D), lambda b, l: (0, 0)),
            pl.BlockSpec((D, H), lambda b, l: (0, 0)),
            pl.BlockSpec((1, H), lambda b, l: (0, 0)),
            pl.BlockSpec((1, H), lambda b, l: (0, 0)),
            pl.BlockSpec((1, H), lambda b, l: (0, 0)),
            pl.BlockSpec((1, 1), lambda b, l: (0, 0)),
            pl.BlockSpec((1, D), lambda b, l: (0, 0)),
            pl.BlockSpec((1, D), lambda b, l: (0, 0)),
        ],
        out_specs=pl.BlockSpec((1, LT, D), lambda b, l: (b, l, 0)),
        out_shape=jax.ShapeDtypeStruct((B, L, D), jnp.float32),
    )(x, g, w3, conf2, W_mem, bm2, Wg1a, wrow, bg12, wg2r, bg22, lng2, lnb2)


def kernel(x, W_key, b_key, cube_keys, cube_values, W_mem, b_mem,
           Wg1, bg1, Wg2, bg2, ln_g, ln_b):
    B, L, D = x.shape
    KD = W_key.shape[1]
    VD = cube_values.shape[1]
    H = Wg1.shape[1]

    q = _pool_q(x, W_key, b_key.reshape(1, KD))
    ti_p, w_p = _sims_topk(q, cube_keys)
    idx = ti_p[:B, :TOPK].reshape(B * TOPK)
    gathered = _gather_rows(cube_values, idx)

    out = _main(
        x,
        gathered.reshape(B, TOPK, VD),
        w_p[:B, :TOPK].reshape(B, 1, TOPK),
        w_p[B:B + 1, 0:1],
        W_mem,
        b_mem.reshape(1, D),
        Wg1[:D],
        Wg1[D:D + 1],
        bg1.reshape(1, H),
        Wg2.reshape(1, H),
        bg2.reshape(1, 1),
        ln_g.reshape(1, D),
        ln_b.reshape(1, D),
    )
    return out


# speedup vs baseline: 1.2058x; 1.1131x over previous
import functools
import math

import jax
import jax.numpy as jnp
from jax import lax
from jax.experimental import pallas as pl
from jax.experimental.pallas import tpu as pltpu

TOPK = 8
LP = 512
LT = 512
SBLK = 5120
NEG = float("-inf")


def _pool_q_kernel(x_ref, wk_ref, bk_ref, q_ref, acc_s):
    i = pl.program_id(0)

    @pl.when(i == 0)
    def _():
        acc_s[...] = jnp.zeros_like(acc_s)

    b = x_ref.shape[0]
    xr = x_ref[...].astype(jnp.bfloat16).astype(jnp.float32)
    acc_s[0:b, :] += jnp.sum(xr, axis=1)

    @pl.when(i == pl.num_programs(0) - 1)
    def _():
        nl = pl.num_programs(0) * x_ref.shape[1]
        xbar = acc_s[0:b, :] * (1.0 / nl)
        wkr = wk_ref[...].astype(jnp.bfloat16).astype(jnp.float32)
        q_ref[...] = (
            jnp.dot(xbar, wkr, preferred_element_type=jnp.float32,
                    precision=lax.Precision.HIGHEST)
            + bk_ref[...]
        )


def _pool_q(x, W_key, b_key2):
    B, L, D = x.shape
    KD = W_key.shape[1]
    return pl.pallas_call(
        _pool_q_kernel,
        grid=(L // LP,),
        in_specs=[
            pl.BlockSpec((B, LP, D), lambda i: (0, i, 0)),
            pl.BlockSpec((D, KD), lambda i: (0, 0)),
            pl.BlockSpec((1, KD), lambda i: (0, 0)),
        ],
        out_specs=pl.BlockSpec((B, KD), lambda i: (0, 0)),
        out_shape=jax.ShapeDtypeStruct((B, KD), jnp.float32),
        scratch_shapes=[pltpu.VMEM((8, D), jnp.float32)],
    )(x, W_key, b_key2)


def _topk_kernel(s_total, q_ref, ck_ref, ti_ref, w_ref, sims_s):
    i = pl.program_id(0)
    B = q_ref.shape[0]
    sblk = ck_ref.shape[0]

    s_blk = lax.dot_general(
        q_ref[...].astype(jnp.bfloat16), ck_ref[...].astype(jnp.bfloat16),
        dimension_numbers=(((1,), (1,)), ((), ())),
        preferred_element_type=jnp.float32,
    )
    gidx = i * sblk + lax.broadcasted_iota(jnp.int32, (B, sblk), 1)
    s_blk = jnp.where(gidx < s_total, s_blk, NEG)
    sims_s[:, pl.ds(i * sblk, sblk)] = s_blk

    @pl.when(i == pl.num_programs(0) - 1)
    def _():
        s = sims_s[...]
        n_pad = s.shape[1]
        iota = lax.broadcasted_iota(jnp.int32, (B, n_pad), 1)
        nv, ni = [], []
        for _ in range(TOPK):
            m = jnp.max(s, axis=1, keepdims=True)
            pos = jnp.min(jnp.where(s == m, iota, n_pad), axis=1,
                          keepdims=True)
            nv.append(m)
            ni.append(pos)
            s = jnp.where(iota == pos, NEG, s)
        tv = jnp.concatenate(nv, axis=1)
        ti = jnp.concatenate(ni, axis=1)
        e = jnp.exp(tv - tv[:, 0:1])
        w = e / jnp.sum(e, axis=1, keepdims=True)
        conf = jnp.mean(w[:, 0:1])
        ti_flat = jnp.concatenate(
            [ti[b:b + 1, :] for b in range(B)], axis=1)
        ti_ref[0:1, 0:B * TOPK] = ti_flat
        w_ref[0:B, 0:TOPK] = w
        w_ref[B:B + 1, 0:1] = jnp.reshape(conf, (1, 1))


def _sims_topk(q, cube_keys):
    B = q.shape[0]
    S, KD = cube_keys.shape
    nblk = (S + SBLK - 1) // SBLK
    return pl.pallas_call(
        functools.partial(_topk_kernel, S),
        grid=(nblk,),
        in_specs=[
            pl.BlockSpec((B, KD), lambda i: (0, 0)),
            pl.BlockSpec((SBLK, KD), lambda i: (i, 0)),
        ],
        out_specs=[
            pl.BlockSpec((8, 128), lambda i: (0, 0)),
            pl.BlockSpec((8, 128), lambda i: (0, 0)),
        ],
        out_shape=[
            jax.ShapeDtypeStruct((8, 128), jnp.int32),
            jax.ShapeDtypeStruct((8, 128), jnp.float32),
        ],
        scratch_shapes=[
            pltpu.VMEM((B, nblk * SBLK), jnp.float32),
        ],
    )(q, cube_keys)


def _gather_rows(cube_values, idx_flat, n):
    from jax.experimental.pallas import tpu_sc as plsc

    VD = cube_values.shape[1]
    mesh = plsc.VectorSubcoreMesh(core_axis_name="c", subcore_axis_name="s")

    @functools.partial(
        pl.kernel,
        mesh=mesh,
        out_type=jax.ShapeDtypeStruct((n, VD), jnp.float32),
        scratch_types=[
            pltpu.VMEM((n,), jnp.int32),
            pltpu.VMEM((n, VD), jnp.float32),
            pltpu.SemaphoreType.DMA,
        ],
    )
    def k(cv_hbm, idx_hbm, out_hbm, idx_v, rows_v, sem):
        wid = lax.axis_index("c") * 16 + lax.axis_index("s")

        @pl.when(wid == 0)
        def _():
            pltpu.sync_copy(idx_hbm.at[pl.ds(0, n)], idx_v)
            pltpu.async_copy(cv_hbm.at[idx_v], rows_v, sem).wait()
            pltpu.sync_copy(rows_v, out_hbm)

    return k(cube_values, idx_flat)


def _main_kernel(x_ref, g_ref, w_ref, c_ref, wm_ref, bm_ref, wg1_ref,
                 bg1_ref, wg2_ref, bg2_ref, lng_ref, lnb_ref,
                 out_ref, mem_s):
    l = pl.program_id(1)
    d = x_ref.shape[2]
    xb = x_ref[0]
    conf = c_ref[0, 0, 0]

    @pl.when(l == 0)
    def _():
        mv = jnp.dot(w_ref[0, :, 0:TOPK], g_ref[0],
                     preferred_element_type=jnp.float32,
                     precision=lax.Precision.HIGHEST)
        mem_s[0:1, :] = (
            jnp.dot(mv, wm_ref[...], preferred_element_type=jnp.float32,
                    precision=lax.Precision.HIGHEST)
            + bm_ref[...]
        )

    mem = mem_s[0:1, :]
    h = (
        jnp.dot(xb.astype(jnp.bfloat16),
                wg1_ref[0:d, :].astype(jnp.bfloat16),
                preferred_element_type=jnp.float32)
        + conf * wg1_ref[d:d + 1, :]
        + bg1_ref[...]
    )
    h = 0.5 * h * (1.0 + lax.erf(h * (1.0 / math.sqrt(2.0))))
    ap = jnp.sum(h * wg2_ref[...], axis=1, keepdims=True) + bg2_ref[...]
    alpha = jax.nn.sigmoid(ap)

    y = xb + (1.0 - alpha) * mem
    mu = jnp.mean(y, axis=1, keepdims=True)
    yc = y - mu
    var = jnp.mean(yc * yc, axis=1, keepdims=True)
    out_ref[0] = yc * lax.rsqrt(var + 1e-5) * lng_ref[...] + lnb_ref[...]


def _main(x, g3, w3, W_mem, bm2, Wg1, bg12, wg2r, bg22, lng2, lnb2):
    B, L, D = x.shape
    K, VD = g3.shape[1], g3.shape[2]
    H = Wg1.shape[1]
    return pl.pallas_call(
        _main_kernel,
        grid=(B, L // LT),
        in_specs=[
            pl.BlockSpec((1, LT, D), lambda b, l: (b, l, 0)),
            pl.BlockSpec((1, K, VD), lambda b, l: (b, 0, 0)),
            pl.BlockSpec((1, 1, 128), lambda b, l: (b, 0, 0)),
            pl.BlockSpec((1, 1, 128), lambda b, l: (B, 0, 0)),
            pl.BlockSpec((VD, D), lambda b, l: (0, 0)),
            pl.BlockSpec((1, D), lambda b, l: (0, 0)),
            pl.BlockSpec((D + 1, H), lambda b, l: (0, 0)),
            pl.BlockSpec((1, H), lambda b, l: (0, 0)),
            pl.BlockSpec((1, H), lambda b, l: (0, 0)),
            pl.BlockSpec((1, 1), lambda b, l: (0, 0)),
            pl.BlockSpec((1, D), lambda b, l: (0, 0)),
            pl.BlockSpec((1, D), lambda b, l: (0, 0)),
        ],
        out_specs=pl.BlockSpec((1, LT, D), lambda b, l: (b, l, 0)),
        out_shape=jax.ShapeDtypeStruct((B, L, D), jnp.float32),
        scratch_shapes=[pltpu.VMEM((8, D), jnp.float32)],
    )(x, g3, w3, w3, W_mem, bm2, Wg1, bg12, wg2r, bg22, lng2, lnb2)


def kernel(x, W_key, b_key, cube_keys, cube_values, W_mem, b_mem,
           Wg1, bg1, Wg2, bg2, ln_g, ln_b):
    B, L, D = x.shape
    KD = W_key.shape[1]
    VD = cube_values.shape[1]
    H = Wg1.shape[1]

    q = _pool_q(x, W_key, b_key.reshape(1, KD))
    ti_p, w_p = _sims_topk(q, cube_keys)
    gathered = _gather_rows(cube_values, ti_p.reshape(8 * 128),
                            B * TOPK)

    out = _main(
        x,
        gathered.reshape(B, TOPK, VD),
        w_p.reshape(8, 1, 128),
        W_mem,
        b_mem.reshape(1, D),
        Wg1,
        bg1.reshape(1, H),
        Wg2.reshape(1, H),
        bg2.reshape(1, 1),
        ln_g.reshape(1, D),
        ln_b.reshape(1, D),
    )
    return out
